# baseline (speedup 1.0000x reference)
import jax
import jax.numpy as jnp
from jax.experimental import pallas as pl

N_CLASSES = 150
EPS = 0.1
ON = 1.0 - EPS
OFF = EPS / (N_CLASSES - 1)

N, H, W = 8, 224, 224
HW = H * W
S = 6272


def _tc_body(lab_ref, out_ref):
    lab = lab_ref[0]
    cls = jax.lax.broadcasted_iota(jnp.int32, (N_CLASSES, S), 0)
    out_ref[0] = jnp.where(cls == lab, ON, OFF)


def kernel(labels):
    lab = labels.astype(jnp.int32).reshape(N, 1, HW)
    out = pl.pallas_call(
        _tc_body,
        grid=(N, HW // S),
        in_specs=[pl.BlockSpec((1, 1, S), lambda n, j: (n, 0, j))],
        out_specs=pl.BlockSpec((1, N_CLASSES, S), lambda n, j: (n, 0, j)),
        out_shape=jax.ShapeDtypeStruct((N, N_CLASSES, HW), jnp.float32),
    )(lab)
    return out.reshape(N, N_CLASSES, H, W)


# speedup vs baseline: 1.0176x; 1.0176x over previous
import jax
import jax.numpy as jnp
from jax.experimental import pallas as pl

N_CLASSES = 150
EPS = 0.1
ON = 1.0 - EPS
OFF = EPS / (N_CLASSES - 1)

N, H, W = 8, 224, 224
HW = H * W
CB = 32


def _tc_body(lab_ref, out_ref):
    j = pl.program_id(1)
    lab = lab_ref[0]
    cls = jax.lax.broadcasted_iota(jnp.int32, (CB, HW), 0) + j * CB
    out_ref[0] = jnp.where(cls == lab, ON, OFF)


def kernel(labels):
    lab = labels.astype(jnp.int32).reshape(N, 1, HW)
    out = pl.pallas_call(
        _tc_body,
        grid=(N, pl.cdiv(N_CLASSES, CB)),
        in_specs=[pl.BlockSpec((1, 1, HW), lambda n, j: (n, 0, 0))],
        out_specs=pl.BlockSpec((1, CB, HW), lambda n, j: (n, j, 0)),
        out_shape=jax.ShapeDtypeStruct((N, N_CLASSES, HW), jnp.float32),
    )(lab)
    return out.reshape(N, N_CLASSES, H, W)


# speedup vs baseline: 4.9828x; 4.8969x over previous
import jax
import jax.numpy as jnp
from jax.experimental import pallas as pl

N_CLASSES = 150
EPS = 0.1
ON = 1.0 - EPS
OFF = EPS / (N_CLASSES - 1)

N, H, W = 8, 224, 224
HW = H * W
CB = 32


def _tc_body(lab_ref, out_ref):
    j = pl.program_id(1)
    lab = lab_ref[0]
    cls = jax.lax.broadcasted_iota(jnp.int32, (CB, H, W), 0) + j * CB
    out_ref[0] = jnp.where(cls == lab, ON, OFF)


def kernel(labels):
    lab = labels.astype(jnp.int32).reshape(N, 1, H, W)
    out = pl.pallas_call(
        _tc_body,
        grid=(N, pl.cdiv(N_CLASSES, CB)),
        in_specs=[pl.BlockSpec((1, 1, H, W), lambda n, j: (n, 0, 0, 0))],
        out_specs=pl.BlockSpec((1, CB, H, W), lambda n, j: (n, j, 0, 0)),
        out_shape=jax.ShapeDtypeStruct((N, N_CLASSES, H, W), jnp.float32),
    )(lab)
    return out


# speedup vs baseline: 5.0310x; 1.0097x over previous
import jax
import jax.numpy as jnp
from jax.experimental import pallas as pl

N_CLASSES = 150
EPS = 0.1
ON = 1.0 - EPS
OFF = EPS / (N_CLASSES - 1)

N, H, W = 8, 224, 224
HW = H * W
CB = 75


def _tc_body(lab_ref, out_ref):
    j = pl.program_id(1)
    lab = lab_ref[0]
    cls = jax.lax.broadcasted_iota(jnp.int32, (CB, H, W), 0) + j * CB
    out_ref[0] = jnp.where(cls == lab, ON, OFF)


def kernel(labels):
    lab = labels.astype(jnp.int32).reshape(N, 1, H, W)
    out = pl.pallas_call(
        _tc_body,
        grid=(N, pl.cdiv(N_CLASSES, CB)),
        in_specs=[pl.BlockSpec((1, 1, H, W), lambda n, j: (n, 0, 0, 0))],
        out_specs=pl.BlockSpec((1, CB, H, W), lambda n, j: (n, j, 0, 0)),
        out_shape=jax.ShapeDtypeStruct((N, N_CLASSES, H, W), jnp.float32),
    )(lab)
    return out
